# baseline (speedup 1.0000x reference)
import functools

import jax
import jax.numpy as jnp
from jax.experimental import pallas as pl
from jax.experimental.pallas import tpu as pltpu

EPS = 1e-7
MIN_NORM = 1e-15
MAX_NORM = 1e6

_HI = jax.lax.Precision.HIGHEST


def _dot(a, b, dims):
    return jax.lax.dot_general(a, b, dims, precision=_HI,
                               preferred_element_type=jnp.float32)


def _mm(a, b):
    return _dot(a, b, (((1,), (0,)), ((), ())))


def _mmT(a, b):
    return _dot(a, b, (((1,), (1,)), ((), ())))


def _bf(v):
    return v.astype(jnp.bfloat16).astype(jnp.float32)


def _mm_d(a, b):
    return jax.lax.dot_general(a.astype(jnp.bfloat16), b.astype(jnp.bfloat16),
                               (((1,), (0,)), ((), ())),
                               preferred_element_type=jnp.float32)


def _lane(x):
    return jax.lax.broadcasted_iota(jnp.int32, x.shape, x.ndim - 1)


def _spatial(x):
    return jnp.where(_lane(x) == 0, 0.0, x)


def _with_first(rest, first):
    return jnp.where(_lane(rest) == 0, first, rest)


def _neg0(x):
    return jnp.where(_lane(x) == 0, -x, x)


def _first(x):
    return x[..., 0:1]


def _safe_norm(x):
    return jnp.sqrt(jnp.clip(jnp.sum(x * x, -1, keepdims=True), 1e-30, None))


def _arcosh(v):
    return jnp.log(v + jnp.sqrt(jnp.clip(v * v - 1.0, 0.0, None)))


def _mink_dot_rows(x, y):
    return jnp.sum(x * y, -1, keepdims=True) - 2.0 * _first(x) * _first(y)


def _mink_norm(u):
    return jnp.sqrt(jnp.clip(_mink_dot_rows(u, u), EPS, None))


def _proj(x):
    y = _spatial(x)
    first = jnp.sqrt(jnp.clip(1.0 + jnp.sum(y * y, -1, keepdims=True), EPS, None))
    return _with_first(y, first)


def _proj_tan(u, x):
    ux = jnp.sum(_spatial(x) * _spatial(u), -1, keepdims=True)
    first = ux / jnp.clip(_first(x), MIN_NORM, None)
    return _with_first(_spatial(u), first)


def _sinh(t):
    return 0.5 * (jnp.exp(t) - jnp.exp(-t))


def _cosh(t):
    return 0.5 * (jnp.exp(t) + jnp.exp(-t))


def _expmap0(u):
    xr = _spatial(u)
    x_norm = _safe_norm(xr)
    rest = _sinh(x_norm) * xr / x_norm
    return _proj(rest)


def _logmap0(x):
    y = _spatial(x)
    y_norm = _safe_norm(y)
    theta = jnp.clip(_first(x), 1.0 + EPS, None)
    return _arcosh(theta) * y / y_norm


def _expmap(u, x):
    normu = jnp.clip(_mink_norm(u), None, MAX_NORM)
    theta = jnp.clip(normu, MIN_NORM, None)
    res = _cosh(theta) * x + _sinh(theta) * u / theta
    return _proj(res)


def _ptransp0(x, u):
    x0 = _first(x)
    y = _spatial(x)
    y_norm = _safe_norm(y)
    y_unit = y / y_norm
    v = _with_first((1.0 - x0) * y_unit, -y_norm)
    alpha = jnp.sum(y_unit * _spatial(u), -1, keepdims=True)
    return _proj_tan(u - alpha * v, x)


def _mobius_add(x, y):
    u = _logmap0(y)
    v = _ptransp0(x, u)
    return _expmap(v, x)


def _silu(x):
    return x * jax.nn.sigmoid(x)


def _layer(X, dist3, edge3, eye3, gm, n, pW, pbias, lng, lnb, wr, wc, wdp,
           b1, w2, b2):
    d = X.shape[-1]
    xt = _logmap0(X)
    X = _expmap0(_spatial(_mm_d(xt, pW)))
    hyp_bias = _expmap0(_spatial(pbias))
    X = _mobius_add(X, hyp_bias)
    xt2 = _logmap0(X)
    X3 = X.reshape(gm, n, d)
    Xn3 = _neg0(X).reshape(gm, n, d)
    g3 = jnp.stack([_mmT(X3[i], Xn3[i]) for i in range(gm)])
    arc = _arcosh(jnp.clip(-g3, 1.0 + EPS, None))
    geo = jnp.clip(arc * arc, None, 50.0)
    sqrt_geo = jnp.clip(arc, None, 50.0 ** 0.5)
    pr = _mm_d(xt2, wr).reshape(gm, n, d)
    pcb = (_mm_d(xt2, wc) + b1).reshape(gm, n, d)
    S = jnp.stack([dist3, geo], axis=2)
    term = jax.lax.dot_general(S.astype(jnp.bfloat16),
                               wdp.astype(jnp.bfloat16),
                               (((2,), (0,)), ((), ())),
                               preferred_element_type=jnp.float32)
    att = pr[:, :, None, :] + pcb[:, None, :, :] + term
    e = jnp.sum(_bf(_silu(att)) * _bf(w2), axis=-1)
    a = jax.nn.sigmoid(e + b2) * edge3
    c3 = jnp.clip(g3 + 1.0, None, -EPS) - 1.0
    gdiag = jnp.where(eye3, g3, 0.0)
    gii = jnp.sum(gdiag, axis=2, keepdims=True)
    gjj = jnp.sum(gdiag, axis=1, keepdims=True)
    rnormu = jax.lax.rsqrt(jnp.clip(gjj + 2.0 * c3 * g3 + c3 * c3 * gii,
                                    EPS, None))
    s3 = a * sqrt_geo * rnormu
    t3 = jnp.sum(s3 * c3, axis=2, keepdims=True)
    agg3 = jnp.stack([_mm(s3[i], X3[i]) for i in range(gm)])
    agg = agg3.reshape(gm * n, d) + t3.reshape(gm * n, 1) * X
    out = _proj_tan(agg, X) / 100.0
    support = _proj_tan(out, X)
    X = _expmap(support, X)
    hh = _logmap0(X)
    mu = jnp.sum(hh, -1, keepdims=True) / (d - 1)
    dd = _spatial(hh - mu)
    var = jnp.sum(dd * dd, -1, keepdims=True) / (d - 1)
    X = _expmap0(dd / jnp.sqrt(var + 1e-5) * lng + lnb)
    X = _expmap0(_spatial(_silu(_logmap0(X))))
    return X


def _fused(pos_ref, mcol_ref, mrow_ref, anum_ref, u0_ref, emb_ref,
           pW_ref, bias_ref, lng_ref, lnb_ref, wr_ref, wc_ref, wd_ref,
           b1_ref, w2_ref, b2_ref, cent_ref, cw1_ref, cb1_ref, cw2_ref,
           cb2_ref, loss_ref, mae_ref, *, n_layers, n_real, n_mol):
    b = pl.program_id(0)
    nb = pl.num_programs(0)

    cent = _expmap0(_spatial(cent_ref[...]))
    cent_neg = _neg0(cent)

    pos = pos_ref[...]
    gm, n = pos.shape[0], pos.shape[2]
    pos = pos - jnp.sum(pos, axis=2, keepdims=True) / n_real
    d2 = 0.0
    for c in range(3):
        pc = pos[:, c, :]
        dc = pc[:, :, None] - pc[:, None, :]
        d2 = d2 + dc * dc
    dist3 = jnp.sqrt(jnp.clip(d2, 1e-12, None))
    mcol = mcol_ref[...]
    mrow = mrow_ref[...]
    mm3 = mcol * mrow
    dist3 = jnp.where(mm3 != 0.0, dist3, 0.0)
    edge3 = (dist3 <= 5.0).astype(jnp.float32) * mm3

    ii = jax.lax.broadcasted_iota(jnp.int32, (1, n, n), 1)
    jj = jax.lax.broadcasted_iota(jnp.int32, (1, n, n), 2)
    eye3 = ii == jj

    anum = anum_ref[...]
    onehot = (jax.lax.broadcasted_iota(jnp.int32, (gm, n, 10), 2) == anum
              ).astype(jnp.float32)
    emb = _mm(onehot.reshape(gm * n, 10), emb_ref[...])
    X = _expmap0(_spatial(emb))

    for l in range(n_layers):
        X = _layer(X, dist3, edge3, eye3, gm, n,
                   pW_ref[l], bias_ref[l], lng_ref[l], lnb_ref[l],
                   wr_ref[l], wc_ref[l], wd_ref[l],
                   b1_ref[l], w2_ref[l], b2_ref[l, 0, 0])

    m = _mmT(X, cent_neg)
    sq = jnp.clip(_arcosh(jnp.clip(-m, 1.0 + EPS, None)) ** 2, None, 50.0)
    mcol2 = mcol.reshape(gm * n, 1)
    ncd = jnp.sqrt(sq) * mcol2
    o1 = _silu(_mm_d(ncd, cw1_ref[...]) + cb1_ref[...])
    o = _mm_d(o1, cw2_ref[...]) + cb2_ref[0, 0]
    o = o * mcol2
    ob = jnp.sum(o.reshape(gm, n, 1), axis=1, keepdims=True)
    diff = ob - u0_ref[...]
    lpart = jnp.sum(diff * diff).reshape(1, 1)
    mpart = jnp.sum(jnp.abs(diff)).reshape(1, 1)

    @pl.when(b == 0)
    def _init():
        loss_ref[...] = jnp.zeros_like(loss_ref)
        mae_ref[...] = jnp.zeros_like(mae_ref)

    loss_ref[...] += lpart
    mae_ref[...] += mpart

    @pl.when(b == nb - 1)
    def _final():
        loss_ref[...] = jnp.sqrt(loss_ref[...] / n_mol)
        mae_ref[...] = mae_ref[...] / n_mol


def kernel(positions, atom_mask, u0, params, atomic_numbers):
    B, N = atomic_numbers.shape
    D = 128
    NP = 32
    G = 4
    assert B % G == 0
    layers = params['layers']
    L = len(layers)
    f32 = jnp.float32

    def stack(fn):
        return jnp.stack([fn(p) for p in layers]).astype(f32)

    def padW(p):
        w = p['linW'].T
        return jnp.pad(w, ((0, D - w.shape[0]), (0, 0)))

    zero1 = jnp.zeros((1,), f32)
    pW_all = stack(padW)
    bias_all = stack(lambda p: p['bias'])
    lng_all = stack(lambda p: jnp.concatenate([zero1, p['ln_g']])[None, :])
    lnb_all = stack(lambda p: jnp.concatenate([zero1, p['ln_b']])[None, :])
    wr_all = stack(lambda p: p['att_w1'][:, :D].T)
    wc_all = stack(lambda p: p['att_w1'][:, D:2 * D].T)
    wd_all = stack(lambda p: p['att_w1'][:, 2 * D:2 * D + 2].T)
    b1_all = stack(lambda p: p['att_b1'][None, :])
    w2_all = stack(lambda p: p['att_w2'])
    b2_all = stack(lambda p: p['att_b2'][None, :])

    emb_pad = jnp.pad(params['embedding'].astype(f32),
                      ((0, 0), (0, D - params['embedding'].shape[1])))
    cent = params['centroid'].astype(f32)
    cw1 = params['cout_w1'].T.astype(f32)
    cb1 = params['cout_b1'][None, :].astype(f32)
    cw2 = params['cout_w2'].T.astype(f32)
    cb2 = params['cout_b2'][None, :].astype(f32)

    rowpad = ((0, 0), (0, NP - N))
    pos = jnp.pad(positions.astype(f32), rowpad + ((0, 0),)).transpose(0, 2, 1)
    maskp = jnp.pad(atom_mask.astype(f32), rowpad)
    mcol = maskp.reshape(B, NP, 1)
    mrow = maskp.reshape(B, 1, NP)
    anum = jnp.pad(atomic_numbers.astype(jnp.int32), rowpad).reshape(B, NP, 1)
    u0r = u0.astype(f32).reshape(B, 1, 1)

    def bmol(*shape):
        return pl.BlockSpec((G,) + shape, lambda b: (b,) + (0,) * len(shape))

    def whole(a):
        nd = a.ndim
        return pl.BlockSpec(a.shape, lambda b, _n=nd: (0,) * _n)

    consts = [emb_pad, pW_all, bias_all, lng_all, lnb_all, wr_all, wc_all,
              wd_all, b1_all, w2_all, b2_all, cent, cw1, cb1, cw2, cb2]

    out = pl.pallas_call(
        functools.partial(_fused, n_layers=L, n_real=N, n_mol=B),
        grid=(B // G,),
        in_specs=[bmol(3, NP), bmol(NP, 1), bmol(1, NP), bmol(NP, 1),
                  bmol(1, 1)] + [whole(a) for a in consts],
        out_specs=[pl.BlockSpec((1, 1), lambda b: (0, 0)),
                   pl.BlockSpec((1, 1), lambda b: (0, 0))],
        out_shape=[jax.ShapeDtypeStruct((1, 1), f32),
                   jax.ShapeDtypeStruct((1, 1), f32)],
    )(pos, mcol, mrow, anum, u0r, *consts)

    loss = out[0][0, 0]
    mae = out[1][0, 0]
    return loss, mae


# speedup vs baseline: 1.0409x; 1.0409x over previous
import functools

import jax
import jax.numpy as jnp
from jax.experimental import pallas as pl
from jax.experimental.pallas import tpu as pltpu

EPS = 1e-7
MIN_NORM = 1e-15
MAX_NORM = 1e6

_HI = jax.lax.Precision.HIGHEST


def _dot(a, b, dims):
    return jax.lax.dot_general(a, b, dims, precision=_HI,
                               preferred_element_type=jnp.float32)


def _mm(a, b):
    return _dot(a, b, (((1,), (0,)), ((), ())))


def _mmT(a, b):
    return _dot(a, b, (((1,), (1,)), ((), ())))


def _bf(v):
    return v.astype(jnp.bfloat16).astype(jnp.float32)


def _mm_d(a, b):
    return jax.lax.dot_general(a.astype(jnp.bfloat16), b.astype(jnp.bfloat16),
                               (((1,), (0,)), ((), ())),
                               preferred_element_type=jnp.float32)


def _lane(x):
    return jax.lax.broadcasted_iota(jnp.int32, x.shape, x.ndim - 1)


def _spatial(x):
    return jnp.where(_lane(x) == 0, 0.0, x)


def _with_first(rest, first):
    return jnp.where(_lane(rest) == 0, first, rest)


def _neg0(x):
    return jnp.where(_lane(x) == 0, -x, x)


def _first(x):
    return x[..., 0:1]


def _safe_norm(x):
    return jnp.sqrt(jnp.clip(jnp.sum(x * x, -1, keepdims=True), 1e-30, None))


def _arcosh(v):
    return jnp.log(v + jnp.sqrt(jnp.clip(v * v - 1.0, 0.0, None)))


def _mink_dot_rows(x, y):
    return jnp.sum(x * y, -1, keepdims=True) - 2.0 * _first(x) * _first(y)


def _mink_norm(u):
    return jnp.sqrt(jnp.clip(_mink_dot_rows(u, u), EPS, None))


def _proj(x):
    y = _spatial(x)
    first = jnp.sqrt(jnp.clip(1.0 + jnp.sum(y * y, -1, keepdims=True), EPS, None))
    return _with_first(y, first)


def _proj_tan(u, x):
    ux = jnp.sum(_spatial(x) * _spatial(u), -1, keepdims=True)
    first = ux / jnp.clip(_first(x), MIN_NORM, None)
    return _with_first(_spatial(u), first)


def _sinh(t):
    return 0.5 * (jnp.exp(t) - jnp.exp(-t))


def _cosh(t):
    return 0.5 * (jnp.exp(t) + jnp.exp(-t))


def _expmap0(u):
    xr = _spatial(u)
    x_norm = _safe_norm(xr)
    rest = _sinh(x_norm) * xr / x_norm
    return _proj(rest)


def _logmap0(x):
    y = _spatial(x)
    y_norm = _safe_norm(y)
    theta = jnp.clip(_first(x), 1.0 + EPS, None)
    return _arcosh(theta) * y / y_norm


def _expmap(u, x):
    normu = jnp.clip(_mink_norm(u), None, MAX_NORM)
    theta = jnp.clip(normu, MIN_NORM, None)
    res = _cosh(theta) * x + _sinh(theta) * u / theta
    return _proj(res)


def _ptransp0(x, u):
    x0 = _first(x)
    y = _spatial(x)
    y_norm = _safe_norm(y)
    y_unit = y / y_norm
    v = _with_first((1.0 - x0) * y_unit, -y_norm)
    alpha = jnp.sum(y_unit * _spatial(u), -1, keepdims=True)
    return _proj_tan(u - alpha * v, x)


def _mobius_add(x, y):
    u = _logmap0(y)
    v = _ptransp0(x, u)
    return _expmap(v, x)


def _silu(x):
    return x * jax.nn.sigmoid(x)


def _layer(X, bdist4, edge3, eye3, gm, n, pW, pbias, lng, lnb, wr, wc, wdp,
           b1, w2, b2):
    d = X.shape[-1]
    xt = _logmap0(X)
    X = _expmap0(_spatial(_mm_d(xt, pW)))
    hyp_bias = _expmap0(_spatial(pbias))
    X = _mobius_add(X, hyp_bias)
    xt2 = _logmap0(X)
    X3 = X.reshape(gm, n, d)
    Xn3 = _neg0(X).reshape(gm, n, d)
    g3 = jnp.stack([_mmT(X3[i], Xn3[i]) for i in range(gm)])
    arc = _arcosh(jnp.clip(-g3, 1.0 + EPS, None))
    geo = jnp.clip(arc * arc, None, 50.0)
    sqrt_geo = jnp.clip(arc, None, 50.0 ** 0.5)
    pr = _mm_d(xt2, wr).reshape(gm, n, d)
    pcb = (_mm_d(xt2, wc) + b1).reshape(gm, n, d)
    att = pr[:, :, None, :] + pcb[:, None, :, :] \
        + bdist4 * _bf(wdp[0:1, :]) \
        + _bf(geo)[..., None] * _bf(wdp[1:2, :])
    e = jnp.sum(_bf(_silu(att)) * _bf(w2), axis=-1)
    a = jax.nn.sigmoid(e + b2) * edge3
    c3 = jnp.clip(g3 + 1.0, None, -EPS) - 1.0
    gdiag = jnp.where(eye3, g3, 0.0)
    gii = jnp.sum(gdiag, axis=2, keepdims=True)
    gjj = jnp.sum(gdiag, axis=1, keepdims=True)
    rnormu = jax.lax.rsqrt(jnp.clip(gjj + 2.0 * c3 * g3 + c3 * c3 * gii,
                                    EPS, None))
    s3 = a * sqrt_geo * rnormu
    t3 = jnp.sum(s3 * c3, axis=2, keepdims=True)
    agg3 = jnp.stack([_mm(s3[i], X3[i]) for i in range(gm)])
    agg = agg3.reshape(gm * n, d) + t3.reshape(gm * n, 1) * X
    out = _proj_tan(agg, X) / 100.0
    support = _proj_tan(out, X)
    X = _expmap(support, X)
    hh = _logmap0(X)
    mu = jnp.sum(hh, -1, keepdims=True) / (d - 1)
    dd = _spatial(hh - mu)
    var = jnp.sum(dd * dd, -1, keepdims=True) / (d - 1)
    X = _expmap0(dd / jnp.sqrt(var + 1e-5) * lng + lnb)
    X = _expmap0(_spatial(_silu(_logmap0(X))))
    return X


def _fused(pos_ref, mcol_ref, mrow_ref, anum_ref, u0_ref, emb_ref,
           pW_ref, bias_ref, lng_ref, lnb_ref, wr_ref, wc_ref, wd_ref,
           b1_ref, w2_ref, b2_ref, cent_ref, cw1_ref, cb1_ref, cw2_ref,
           cb2_ref, loss_ref, mae_ref, *, n_layers, n_real, n_mol):
    b = pl.program_id(0)
    nb = pl.num_programs(0)

    cent = _expmap0(_spatial(cent_ref[...]))
    cent_neg = _neg0(cent)

    pos = pos_ref[...]
    gm, n = pos.shape[0], pos.shape[2]
    pos = pos - jnp.sum(pos, axis=2, keepdims=True) / n_real
    d2 = 0.0
    for c in range(3):
        pc = pos[:, c, :]
        dc = pc[:, :, None] - pc[:, None, :]
        d2 = d2 + dc * dc
    dist3 = jnp.sqrt(jnp.clip(d2, 1e-12, None))
    mcol = mcol_ref[...]
    mrow = mrow_ref[...]
    mm3 = mcol * mrow
    dist3 = jnp.where(mm3 != 0.0, dist3, 0.0)
    edge3 = (dist3 <= 5.0).astype(jnp.float32) * mm3
    bdist4 = _bf(dist3)[..., None]

    ii = jax.lax.broadcasted_iota(jnp.int32, (1, n, n), 1)
    jj = jax.lax.broadcasted_iota(jnp.int32, (1, n, n), 2)
    eye3 = ii == jj

    anum = anum_ref[...]
    onehot = (jax.lax.broadcasted_iota(jnp.int32, (gm, n, 10), 2) == anum
              ).astype(jnp.float32)
    emb = _mm(onehot.reshape(gm * n, 10), emb_ref[...])
    X = _expmap0(_spatial(emb))

    for l in range(n_layers):
        X = _layer(X, bdist4, edge3, eye3, gm, n,
                   pW_ref[l], bias_ref[l], lng_ref[l], lnb_ref[l],
                   wr_ref[l], wc_ref[l], wd_ref[l],
                   b1_ref[l], w2_ref[l], b2_ref[l, 0, 0])

    m = _mmT(X, cent_neg)
    sq = jnp.clip(_arcosh(jnp.clip(-m, 1.0 + EPS, None)) ** 2, None, 50.0)
    mcol2 = mcol.reshape(gm * n, 1)
    ncd = jnp.sqrt(sq) * mcol2
    o1 = _silu(_mm_d(ncd, cw1_ref[...]) + cb1_ref[...])
    o = _mm_d(o1, cw2_ref[...]) + cb2_ref[0, 0]
    o = o * mcol2
    ob = jnp.sum(o.reshape(gm, n, 1), axis=1, keepdims=True)
    diff = ob - u0_ref[...]
    lpart = jnp.sum(diff * diff).reshape(1, 1)
    mpart = jnp.sum(jnp.abs(diff)).reshape(1, 1)

    @pl.when(b == 0)
    def _init():
        loss_ref[...] = jnp.zeros_like(loss_ref)
        mae_ref[...] = jnp.zeros_like(mae_ref)

    loss_ref[...] += lpart
    mae_ref[...] += mpart

    @pl.when(b == nb - 1)
    def _final():
        loss_ref[...] = jnp.sqrt(loss_ref[...] / n_mol)
        mae_ref[...] = mae_ref[...] / n_mol


def kernel(positions, atom_mask, u0, params, atomic_numbers):
    B, N = atomic_numbers.shape
    D = 128
    NP = 32
    G = 8
    assert B % G == 0
    layers = params['layers']
    L = len(layers)
    f32 = jnp.float32

    def stack(fn):
        return jnp.stack([fn(p) for p in layers]).astype(f32)

    def padW(p):
        w = p['linW'].T
        return jnp.pad(w, ((0, D - w.shape[0]), (0, 0)))

    zero1 = jnp.zeros((1,), f32)
    pW_all = stack(padW)
    bias_all = stack(lambda p: p['bias'])
    lng_all = stack(lambda p: jnp.concatenate([zero1, p['ln_g']])[None, :])
    lnb_all = stack(lambda p: jnp.concatenate([zero1, p['ln_b']])[None, :])
    wr_all = stack(lambda p: p['att_w1'][:, :D].T)
    wc_all = stack(lambda p: p['att_w1'][:, D:2 * D].T)
    wd_all = stack(lambda p: p['att_w1'][:, 2 * D:2 * D + 2].T)
    b1_all = stack(lambda p: p['att_b1'][None, :])
    w2_all = stack(lambda p: p['att_w2'])
    b2_all = stack(lambda p: p['att_b2'][None, :])

    emb_pad = jnp.pad(params['embedding'].astype(f32),
                      ((0, 0), (0, D - params['embedding'].shape[1])))
    cent = params['centroid'].astype(f32)
    cw1 = params['cout_w1'].T.astype(f32)
    cb1 = params['cout_b1'][None, :].astype(f32)
    cw2 = params['cout_w2'].T.astype(f32)
    cb2 = params['cout_b2'][None, :].astype(f32)

    rowpad = ((0, 0), (0, NP - N))
    pos = jnp.pad(positions.astype(f32), rowpad + ((0, 0),)).transpose(0, 2, 1)
    maskp = jnp.pad(atom_mask.astype(f32), rowpad)
    mcol = maskp.reshape(B, NP, 1)
    mrow = maskp.reshape(B, 1, NP)
    anum = jnp.pad(atomic_numbers.astype(jnp.int32), rowpad).reshape(B, NP, 1)
    u0r = u0.astype(f32).reshape(B, 1, 1)

    def bmol(*shape):
        return pl.BlockSpec((G,) + shape, lambda b: (b,) + (0,) * len(shape))

    def whole(a):
        nd = a.ndim
        return pl.BlockSpec(a.shape, lambda b, _n=nd: (0,) * _n)

    consts = [emb_pad, pW_all, bias_all, lng_all, lnb_all, wr_all, wc_all,
              wd_all, b1_all, w2_all, b2_all, cent, cw1, cb1, cw2, cb2]

    out = pl.pallas_call(
        functools.partial(_fused, n_layers=L, n_real=N, n_mol=B),
        grid=(B // G,),
        in_specs=[bmol(3, NP), bmol(NP, 1), bmol(1, NP), bmol(NP, 1),
                  bmol(1, 1)] + [whole(a) for a in consts],
        out_specs=[pl.BlockSpec((1, 1), lambda b: (0, 0)),
                   pl.BlockSpec((1, 1), lambda b: (0, 0))],
        out_shape=[jax.ShapeDtypeStruct((1, 1), f32),
                   jax.ShapeDtypeStruct((1, 1), f32)],
    )(pos, mcol, mrow, anum, u0r, *consts)

    loss = out[0][0, 0]
    mae = out[1][0, 0]
    return loss, mae
